# baseline (speedup 1.0000x reference)
import functools
import math

import jax
import jax.numpy as jnp
from jax import lax
from jax.experimental import pallas as pl
from jax.experimental.pallas import tpu as pltpu
from jax.experimental.pallas import tpu_sc as plsc

D_MODEL = 1024
SCALE = math.sqrt(D_MODEL)
NUM_CORES = 2
NUM_SUBCORES = 16
NW = NUM_CORES * NUM_SUBCORES
LANES = 16
CHUNK = 32


@functools.lru_cache(maxsize=None)
def _make_sc_kernel(B):
    assert B % (8 * NW) == 0
    bpw = B // NW
    assert bpw % CHUNK == 0
    nch = bpw // CHUNK
    mesh = plsc.VectorSubcoreMesh(core_axis_name="c", subcore_axis_name="s")

    @functools.partial(
        pl.kernel,
        mesh=mesh,
        out_type=jax.ShapeDtypeStruct((B, D_MODEL), jnp.float32),
        scratch_types=[
            pltpu.VMEM((bpw,), jnp.int32),
            pltpu.VMEM((CHUNK, D_MODEL), jnp.float32),
            pltpu.SemaphoreType.DMA,
        ],
    )
    def emb_kernel(table_hbm, idx_hbm, out_hbm, idx_v, rows_v, sem):
        wid = lax.axis_index("s") * NUM_CORES + lax.axis_index("c")
        base = wid * bpw
        pltpu.sync_copy(idx_hbm.at[pl.ds(base, bpw)], idx_v)

        def chunk_body(ci, carry):
            pltpu.async_copy(
                table_hbm.at[idx_v.at[pl.ds(ci * CHUNK, CHUNK)]], rows_v, sem
            ).wait()

            def row_body(r, c2):
                def vec_body(v, c3):
                    sl = pl.ds(v * LANES, LANES)
                    rows_v[r, sl] = rows_v[r, sl] * SCALE
                    return c3

                return lax.fori_loop(0, D_MODEL // LANES, vec_body, c2)

            lax.fori_loop(0, CHUNK, row_body, 0)
            pltpu.sync_copy(rows_v, out_hbm.at[pl.ds(base + ci * CHUNK, CHUNK)])
            return carry

        lax.fori_loop(0, nch, chunk_body, 0)

    return emb_kernel


def kernel(x, embedding):
    idx = x.reshape(-1).astype(jnp.int32)
    out = _make_sc_kernel(idx.shape[0])(embedding, idx)
    return out.reshape(x.shape + (D_MODEL,))


# speedup vs baseline: 3.4142x; 3.4142x over previous
import functools
import math

import jax
import jax.numpy as jnp
from jax import lax
from jax.experimental import pallas as pl
from jax.experimental.pallas import tpu as pltpu
from jax.experimental.pallas import tpu_sc as plsc

D_MODEL = 1024
SCALE = math.sqrt(D_MODEL)
NUM_CORES = 2
NUM_SUBCORES = 16
NW = NUM_CORES * NUM_SUBCORES
LANES = 16
CHUNK = 16
NBUF = 4


@functools.lru_cache(maxsize=None)
def _make_sc_kernel(B):
    assert B % (8 * NW) == 0
    bpw = B // NW
    nch = bpw // CHUNK
    assert nch % NBUF == 0 and nch // NBUF >= 3
    ngrp = nch // NBUF
    mesh = plsc.VectorSubcoreMesh(core_axis_name="c", subcore_axis_name="s")

    @functools.partial(
        pl.kernel,
        mesh=mesh,
        out_type=jax.ShapeDtypeStruct((B, D_MODEL), jnp.float32),
        scratch_types=[
            pltpu.VMEM((bpw,), jnp.int32),
        ]
        + [pltpu.VMEM((CHUNK, D_MODEL), jnp.float32) for _ in range(NBUF)]
        + [pltpu.SemaphoreType.DMA for _ in range(2 * NBUF)],
    )
    def emb_kernel(table_hbm, idx_hbm, out_hbm, idx_v, *rest):
        bufs = rest[:NBUF]
        gsem = rest[NBUF : 2 * NBUF]
        ssem = rest[2 * NBUF :]
        wid = lax.axis_index("s") * NUM_CORES + lax.axis_index("c")
        base = wid * bpw
        pltpu.sync_copy(idx_hbm.at[pl.ds(base, bpw)], idx_v)

        def gather_copy(ci, b):
            return pltpu.make_async_copy(
                table_hbm.at[idx_v.at[pl.ds(ci * CHUNK, CHUNK)]], bufs[b], gsem[b]
            )

        def store_copy(ci, b):
            return pltpu.make_async_copy(
                bufs[b], out_hbm.at[pl.ds(base + ci * CHUNK, CHUNK)], ssem[b]
            )

        def scale(b):
            def row(r, c):
                for v in range(D_MODEL // LANES):
                    sl = pl.ds(v * LANES, LANES)
                    bufs[b][r, sl] = bufs[b][r, sl] * SCALE
                return c

            lax.fori_loop(0, CHUNK, row, 0)

        def step(ci, b, do_store_wait, do_gather_issue):
            bp = (b + NBUF - 1) % NBUF
            gather_copy(ci, b).wait()
            scale(b)
            store_copy(ci, b).start()
            if do_store_wait:
                store_copy(ci - 1, bp).wait()
            if do_gather_issue:
                gather_copy(ci + NBUF - 1, bp).start()

        for b in range(NBUF):
            gather_copy(b, b).start()

        step(0, 0, False, False)
        for b in range(1, NBUF):
            step(b, b, True, True)

        def group(g, c):
            for b in range(NBUF):
                step(g * NBUF + b, b, True, True)
            return c

        lax.fori_loop(1, ngrp - 1, group, 0)

        ci0 = (ngrp - 1) * NBUF
        step(ci0, 0, True, True)
        for b in range(1, NBUF):
            step(ci0 + b, b, True, False)
        store_copy(nch - 1, NBUF - 1).wait()

    return emb_kernel


def kernel(x, embedding):
    idx = x.reshape(-1).astype(jnp.int32)
    out = _make_sc_kernel(idx.shape[0])(embedding, idx)
    return out.reshape(x.shape + (D_MODEL,))
